# baseline (speedup 1.0000x reference)
import functools

import jax
import jax.numpy as jnp
from jax.experimental import pallas as pl
from jax.experimental.pallas import tpu as pltpu


def _gate_and_scale4(x_ref, w1_ref, w2_ref, o_ref):
    x = x_ref[...]
    pooled = jnp.sum(x, axis=(2, 3), dtype=jnp.float32)
    hidden = jnp.maximum(
        jnp.dot(pooled, w1_ref[...], preferred_element_type=jnp.float32), 0.0)
    gate = jax.nn.sigmoid(
        jnp.dot(hidden, w2_ref[...], preferred_element_type=jnp.float32))
    o_ref[...] = x * gate[:, :, None, None]


@functools.partial(jax.jit, static_argnames=("bt",))
def _se_apply(x, w1, w2, bt=1):
    B, C, H, W = x.shape
    HW = H * W
    Cr = w1.shape[1]

    w1_pre = w1.astype(jnp.float32) * jnp.float32(1.0 / HW)
    w2_f = w2.astype(jnp.float32)

    out = pl.pallas_call(
        _gate_and_scale4,
        out_shape=jax.ShapeDtypeStruct((B, C, H, W), x.dtype),
        grid=(B // bt,),
        in_specs=[
            pl.BlockSpec((bt, C, H, W), lambda i: (i, 0, 0, 0)),
            pl.BlockSpec((C, Cr), lambda i: (0, 0)),
            pl.BlockSpec((Cr, C), lambda i: (0, 0)),
        ],
        out_specs=pl.BlockSpec((bt, C, H, W), lambda i: (i, 0, 0, 0)),
        compiler_params=pltpu.CompilerParams(
            dimension_semantics=("parallel",),
            vmem_limit_bytes=60 * 1024 * 1024,
        ),
    )(x, w1_pre, w2_f)
    return out


def kernel(x, w1, w2):
    return _se_apply(x, w1, w2)


# speedup vs baseline: 12.8607x; 12.8607x over previous
import functools

import jax
import jax.numpy as jnp
from jax.experimental import pallas as pl
from jax.experimental.pallas import tpu as pltpu


def _se_hwbc(x_ref, w1_ref, w2_ref, o_ref):
    x = x_ref[...]
    pooled = jnp.sum(x, axis=0, dtype=jnp.float32)
    hidden = jnp.maximum(
        jnp.dot(pooled, w1_ref[...], preferred_element_type=jnp.float32), 0.0)
    gate = jax.nn.sigmoid(
        jnp.dot(hidden, w2_ref[...], preferred_element_type=jnp.float32))
    o_ref[...] = x * gate[None, :, :]


@functools.partial(jax.jit, static_argnames=("bt",))
def _se_apply(x, w1, w2, bt=8):
    B, C, H, W = x.shape
    HW = H * W
    Cr = w1.shape[1]

    x_t = jnp.transpose(x, (2, 3, 0, 1)).reshape(HW, B, C)
    w1_pre = w1.astype(jnp.float32) * jnp.float32(1.0 / HW)
    w2_f = w2.astype(jnp.float32)

    out_t = pl.pallas_call(
        _se_hwbc,
        out_shape=jax.ShapeDtypeStruct((HW, B, C), x.dtype),
        grid=(B // bt,),
        in_specs=[
            pl.BlockSpec((HW, bt, C), lambda i: (0, i, 0)),
            pl.BlockSpec((C, Cr), lambda i: (0, 0)),
            pl.BlockSpec((Cr, C), lambda i: (0, 0)),
        ],
        out_specs=pl.BlockSpec((HW, bt, C), lambda i: (0, i, 0)),
        compiler_params=pltpu.CompilerParams(
            dimension_semantics=("arbitrary",),
            vmem_limit_bytes=60 * 1024 * 1024,
        ),
    )(x_t, w1_pre, w2_f)
    return out_t.reshape(H, W, B, C).transpose(2, 3, 0, 1)


def kernel(x, w1, w2):
    return _se_apply(x, w1, w2)
